# baseline (device time: 42461 ns/iter reference)
import jax
import jax.numpy as jnp
from jax import lax
from jax.experimental import pallas as pl
from jax.experimental.pallas import tpu as pltpu

N_DEV = 4


def kernel(x, w_mat, scale_x, scale_w):
    m_total, k_shard = x.shape
    k_total, n = w_mat.shape
    m_blk = m_total // N_DEV
    h_rows = k_shard // 2

    send_dtype = jnp.float8_e4m3fn
    w_dtype = jnp.float8_e5m2

    def body(x_ref, w_ref, sx_ref, sw_ref, out_ref,
             stage_ref, recv_ref, wbuf_ref, wsmall_ref,
             send_sems, recv_sems, wdma_sems):
        my = lax.axis_index("i")

        recv_order = (1, 3, 2)
        ks = [my] + [lax.rem(my - o + N_DEV, N_DEV) for o in recv_order]

        def wdma_start(j, slot):
            row = ks[j // 2] * k_shard + (j % 2) * h_rows
            dma = pltpu.make_async_copy(
                w_ref.at[pl.ds(row, h_rows), :],
                wbuf_ref.at[slot],
                wdma_sems.at[slot],
            )
            dma.start()
            return dma

        wdmas = [wdma_start(0, 0), wdma_start(1, 1)]

        barrier_sem = pltpu.get_barrier_semaphore()
        for o in range(1, N_DEV):
            peer = lax.rem(my + o, N_DEV)
            pl.semaphore_signal(barrier_sem, inc=1, device_id=(peer,),
                                device_id_type=pl.DeviceIdType.MESH)
        pl.semaphore_wait(barrier_sem, N_DEV - 1)

        rdmas = []
        for o in range(1, N_DEV):
            dst = lax.rem(my + o, N_DEV)
            stage_ref[o - 1] = x_ref[pl.ds(dst * m_blk, m_blk), :].astype(send_dtype)
            rdma = pltpu.make_async_remote_copy(
                src_ref=stage_ref.at[o - 1],
                dst_ref=recv_ref.at[o - 1],
                send_sem=send_sems.at[o - 1],
                recv_sem=recv_sems.at[o - 1],
                device_id=(dst,),
                device_id_type=pl.DeviceIdType.MESH,
            )
            rdma.start()
            rdmas.append(rdma)

        s = sx_ref[0] * sw_ref[0]

        for i in range(N_DEV):
            for h in range(2):
                j = 2 * i + h
                wdmas[j % 2].wait()
                row = ks[i] * k_shard + h * h_rows
                wsmall_ref[pl.ds(row, h_rows), :] = wbuf_ref[j % 2].astype(w_dtype)
                if j + 2 < 2 * N_DEV:
                    wdmas[j % 2] = wdma_start(j + 2, j % 2)

            if i == 0:
                a = x_ref[pl.ds(my * m_blk, m_blk), :].astype(send_dtype)
            else:
                o = recv_order[i - 1]
                rdmas[o - 1].wait_recv()
                a = recv_ref[o - 1]
            b = wsmall_ref[pl.ds(ks[i] * k_shard, k_shard), :]
            dot = lax.dot_general(a, b, (((1,), (0,)), ((), ())),
                                  preferred_element_type=jnp.float32)
            if i == 0:
                out_ref[...] = dot
            elif i < N_DEV - 1:
                out_ref[...] = out_ref[...] + dot
            else:
                out_ref[...] = jnp.maximum((out_ref[...] + dot) * s, 0.0)

        for o in range(1, N_DEV):
            rdmas[o - 1].wait_send()

    return pl.pallas_call(
        body,
        out_shape=jax.ShapeDtypeStruct((m_blk, n), jnp.float32),
        in_specs=[
            pl.BlockSpec(memory_space=pltpu.VMEM),
            pl.BlockSpec(memory_space=pl.ANY),
            pl.BlockSpec(memory_space=pltpu.SMEM),
            pl.BlockSpec(memory_space=pltpu.SMEM),
        ],
        out_specs=pl.BlockSpec(memory_space=pltpu.VMEM),
        scratch_shapes=[
            pltpu.VMEM((N_DEV - 1, m_blk, k_shard), send_dtype),
            pltpu.VMEM((N_DEV - 1, m_blk, k_shard), send_dtype),
            pltpu.VMEM((2, h_rows, n), jnp.float32),
            pltpu.VMEM((k_total, n), w_dtype),
            pltpu.SemaphoreType.DMA((N_DEV - 1,)),
            pltpu.SemaphoreType.DMA((N_DEV - 1,)),
            pltpu.SemaphoreType.DMA((2,)),
        ],
        compiler_params=pltpu.CompilerParams(
            collective_id=0,
            vmem_limit_bytes=40 * 1024 * 1024,
        ),
    )(x, w_mat, scale_x, scale_w)


# device time: 37361 ns/iter; 1.1365x vs baseline; 1.1365x over previous
import os

import jax
import jax.numpy as jnp
from jax import lax
from jax.experimental import pallas as pl
from jax.experimental.pallas import tpu as pltpu

N_DEV = 4

_MODE = os.environ.get("KERNEL_MODE", "full")


def kernel(x, w_mat, scale_x, scale_w):
    m_total, k_shard = x.shape
    k_total, n = w_mat.shape
    m_blk = m_total // N_DEV
    h_rows = k_shard // 2

    send_dtype = jnp.float8_e4m3fn
    w_dtype = jnp.float8_e5m2

    def body(x_ref, w_ref, sx_ref, sw_ref, out_ref,
             stage_ref, recv_ref, wbuf_ref, wsmall_ref,
             send_sems, recv_sems, wdma_sems):
        my = lax.axis_index("i")

        recv_order = (1, 3, 2)
        ks = [my] + [lax.rem(my - o + N_DEV, N_DEV) for o in recv_order]

        def wdma_start(j, slot):
            row = ks[j // 2] * k_shard + (j % 2) * h_rows
            dma = pltpu.make_async_copy(
                w_ref.at[pl.ds(row, h_rows), :],
                wbuf_ref.at[slot],
                wdma_sems.at[slot],
            )
            dma.start()
            return dma

        if _MODE != "comm":
            wdmas = [wdma_start(0, 0), wdma_start(1, 1)]

        if _MODE != "compute":
            barrier_sem = pltpu.get_barrier_semaphore()
            for o in range(1, N_DEV):
                peer = lax.rem(my + o, N_DEV)
                pl.semaphore_signal(barrier_sem, inc=1, device_id=(peer,),
                                    device_id_type=pl.DeviceIdType.MESH)
            pl.semaphore_wait(barrier_sem, N_DEV - 1)

        rdmas = []
        for o in range(1, N_DEV):
            dst = lax.rem(my + o, N_DEV)
            stage_ref[o - 1] = x_ref[pl.ds(dst * m_blk, m_blk), :].astype(send_dtype)
            if _MODE == "compute":
                continue
            rdma = pltpu.make_async_remote_copy(
                src_ref=stage_ref.at[o - 1],
                dst_ref=recv_ref.at[o - 1],
                send_sem=send_sems.at[o - 1],
                recv_sem=recv_sems.at[o - 1],
                device_id=(dst,),
                device_id_type=pl.DeviceIdType.MESH,
            )
            rdma.start()
            rdmas.append(rdma)

        s = sx_ref[0] * sw_ref[0]

        if _MODE == "comm":
            for o in recv_order:
                rdmas[o - 1].wait_recv()
            out_ref[...] = jnp.broadcast_to(
                recv_ref[0].astype(jnp.float32)[:, :1]
                + recv_ref[1].astype(jnp.float32)[:, :1]
                + recv_ref[2].astype(jnp.float32)[:, :1],
                (m_blk, n),
            )
            for o in range(1, N_DEV):
                rdmas[o - 1].wait_send()
            return

        for i in range(N_DEV):
            for h in range(2):
                j = 2 * i + h
                wdmas[j % 2].wait()
                row = ks[i] * k_shard + h * h_rows
                wsmall_ref[pl.ds(row, h_rows), :] = wbuf_ref[j % 2].astype(w_dtype)
                if j + 2 < 2 * N_DEV:
                    wdmas[j % 2] = wdma_start(j + 2, j % 2)

            if i == 0:
                a = x_ref[pl.ds(my * m_blk, m_blk), :].astype(send_dtype)
            else:
                o = recv_order[i - 1]
                if _MODE == "compute":
                    a = stage_ref[o - 1]
                else:
                    rdmas[o - 1].wait_recv()
                    a = recv_ref[o - 1]
            b = wsmall_ref[pl.ds(ks[i] * k_shard, k_shard), :]
            dot = lax.dot_general(a, b, (((1,), (0,)), ((), ())),
                                  preferred_element_type=jnp.float32)
            if i == 0:
                out_ref[...] = dot
            elif i < N_DEV - 1:
                out_ref[...] = out_ref[...] + dot
            else:
                out_ref[...] = jnp.maximum((out_ref[...] + dot) * s, 0.0)

        if _MODE != "compute":
            for o in range(1, N_DEV):
                rdmas[o - 1].wait_send()

    return pl.pallas_call(
        body,
        out_shape=jax.ShapeDtypeStruct((m_blk, n), jnp.float32),
        in_specs=[
            pl.BlockSpec(memory_space=pltpu.VMEM),
            pl.BlockSpec(memory_space=pl.ANY),
            pl.BlockSpec(memory_space=pltpu.SMEM),
            pl.BlockSpec(memory_space=pltpu.SMEM),
        ],
        out_specs=pl.BlockSpec(memory_space=pltpu.VMEM),
        scratch_shapes=[
            pltpu.VMEM((N_DEV - 1, m_blk, k_shard), send_dtype),
            pltpu.VMEM((N_DEV - 1, m_blk, k_shard), send_dtype),
            pltpu.VMEM((2, h_rows, n), jnp.float32),
            pltpu.VMEM((k_total, n), w_dtype),
            pltpu.SemaphoreType.DMA((N_DEV - 1,)),
            pltpu.SemaphoreType.DMA((N_DEV - 1,)),
            pltpu.SemaphoreType.DMA((2,)),
        ],
        compiler_params=pltpu.CompilerParams(
            collective_id=0,
            vmem_limit_bytes=40 * 1024 * 1024,
        ),
    )(x, w_mat, scale_x, scale_w)


# device time: 26359 ns/iter; 1.6109x vs baseline; 1.4174x over previous
import os

import jax
import jax.numpy as jnp
from jax import lax
from jax.experimental import pallas as pl
from jax.experimental.pallas import tpu as pltpu

N_DEV = 4

_MODE = os.environ.get("KERNEL_MODE", "full")
_SKIP_O = {int(t) for t in os.environ.get("KERNEL_SKIP_O", "").split(",") if t}


def kernel(x, w_mat, scale_x, scale_w):
    m_total, k_shard = x.shape
    k_total, n = w_mat.shape
    m_blk = m_total // N_DEV
    h_rows = k_shard // 2

    send_dtype = jnp.float8_e4m3fn
    w_dtype = jnp.float8_e5m2

    def body(x_ref, w_ref, sx_ref, sw_ref, out_ref,
             stage_ref, recv_ref, wbuf_ref, wsmall_ref,
             send_sems, recv_sems, wdma_sems):
        my = lax.axis_index("i")

        recv_order = (1, 3, 2)
        ks = [my] + [lax.rem(my - o + N_DEV, N_DEV) for o in recv_order]

        def wdma_start(j, slot):
            row = ks[j // 2] * k_shard + (j % 2) * h_rows
            dma = pltpu.make_async_copy(
                w_ref.at[pl.ds(row, h_rows), :],
                wbuf_ref.at[slot],
                wdma_sems.at[slot],
            )
            dma.start()
            return dma

        if _MODE != "comm":
            wdmas = [wdma_start(0, 0), wdma_start(1, 1)]

        if _MODE != "compute":
            barrier_sem = pltpu.get_barrier_semaphore()
            for o in range(1, N_DEV):
                peer = lax.rem(my + o, N_DEV)
                pl.semaphore_signal(barrier_sem, inc=1, device_id=(peer,),
                                    device_id_type=pl.DeviceIdType.MESH)
            pl.semaphore_wait(barrier_sem, N_DEV - 1)

        rdmas = []
        for o in range(1, N_DEV):
            dst = lax.rem(my + o, N_DEV)
            stage_ref[o - 1] = x_ref[pl.ds(dst * m_blk, m_blk), :].astype(send_dtype)
            if _MODE == "compute" or o in _SKIP_O:
                rdmas.append(None)
                continue
            rdma = pltpu.make_async_remote_copy(
                src_ref=stage_ref.at[o - 1],
                dst_ref=recv_ref.at[o - 1],
                send_sem=send_sems.at[o - 1],
                recv_sem=recv_sems.at[o - 1],
                device_id=(dst,),
                device_id_type=pl.DeviceIdType.MESH,
            )
            rdma.start()
            rdmas.append(rdma)

        s = sx_ref[0] * sw_ref[0]

        if _MODE == "comm":
            for o in recv_order:
                if o in _SKIP_O:
                    continue
                rdmas[o - 1].wait_recv()
            out_ref[...] = jnp.broadcast_to(
                recv_ref[0].astype(jnp.float32)[:, :1]
                + recv_ref[1].astype(jnp.float32)[:, :1]
                + recv_ref[2].astype(jnp.float32)[:, :1],
                (m_blk, n),
            )
            for o in range(1, N_DEV):
                if o in _SKIP_O:
                    continue
                rdmas[o - 1].wait_send()
            return

        for i in range(N_DEV):
            for h in range(2):
                j = 2 * i + h
                wdmas[j % 2].wait()
                row = ks[i] * k_shard + h * h_rows
                wsmall_ref[pl.ds(row, h_rows), :] = wbuf_ref[j % 2].astype(w_dtype)
                if j + 2 < 2 * N_DEV:
                    wdmas[j % 2] = wdma_start(j + 2, j % 2)

            if i == 0:
                a = x_ref[pl.ds(my * m_blk, m_blk), :].astype(send_dtype)
            else:
                o = recv_order[i - 1]
                if _MODE == "compute":
                    a = stage_ref[o - 1]
                else:
                    rdmas[o - 1].wait_recv()
                    a = recv_ref[o - 1]
            b = wsmall_ref[pl.ds(ks[i] * k_shard, k_shard), :]
            dot = lax.dot_general(a, b, (((1,), (0,)), ((), ())),
                                  preferred_element_type=jnp.float32)
            if i == 0:
                out_ref[...] = dot
            elif i < N_DEV - 1:
                out_ref[...] = out_ref[...] + dot
            else:
                out_ref[...] = jnp.maximum((out_ref[...] + dot) * s, 0.0)

        if _MODE != "compute":
            for o in range(1, N_DEV):
                rdmas[o - 1].wait_send()

    return pl.pallas_call(
        body,
        out_shape=jax.ShapeDtypeStruct((m_blk, n), jnp.float32),
        in_specs=[
            pl.BlockSpec(memory_space=pltpu.VMEM),
            pl.BlockSpec(memory_space=pl.ANY),
            pl.BlockSpec(memory_space=pltpu.SMEM),
            pl.BlockSpec(memory_space=pltpu.SMEM),
        ],
        out_specs=pl.BlockSpec(memory_space=pltpu.VMEM),
        scratch_shapes=[
            pltpu.VMEM((N_DEV - 1, m_blk, k_shard), send_dtype),
            pltpu.VMEM((N_DEV - 1, m_blk, k_shard), send_dtype),
            pltpu.VMEM((2, h_rows, n), jnp.float32),
            pltpu.VMEM((k_total, n), w_dtype),
            pltpu.SemaphoreType.DMA((N_DEV - 1,)),
            pltpu.SemaphoreType.DMA((N_DEV - 1,)),
            pltpu.SemaphoreType.DMA((2,)),
        ],
        compiler_params=pltpu.CompilerParams(
            collective_id=0,
            vmem_limit_bytes=40 * 1024 * 1024,
        ),
    )(x, w_mat, scale_x, scale_w)
